# baseline (speedup 1.0000x reference)
import functools

import jax
import jax.numpy as jnp
from jax import lax
from jax.experimental import pallas as pl
from jax.experimental.pallas import tpu as pltpu
from jax.experimental.pallas import tpu_sc as plsc

_A_OFF = 0
_B_OFF = 512
_DCMD_OFF = 1024
_DVE_OFF = 1088
_VT_OFF = 2624
_E_OFF = 3136
_ZERO_ROW = 3648
_N_ROWS = 3656

_NW = 32
_B_PER_W = 32
_K = 21
_KP = 24
_D = 128


def _mm(x, y):
    return lax.dot_general(x, y, (((1,), (0,)), ((), ())),
                           precision=lax.Precision.HIGHEST,
                           preferred_element_type=jnp.float32)


def _mmT(x, y):
    return lax.dot_general(x, y, (((1,), (1,)), ((), ())),
                           precision=lax.Precision.HIGHEST,
                           preferred_element_type=jnp.float32)


def _onehot(idx_col, n):
    iota = lax.broadcasted_iota(jnp.int32, (idx_col.shape[0], n), 1)
    return (idx_col == iota).astype(jnp.float32)


def _count_mat(idx2, n, mask=None):
    r, j = idx2.shape
    if mask is not None:
        idx2 = jnp.where(mask, idx2, n)
    oh = (idx2[:, :, None] ==
          lax.broadcasted_iota(jnp.int32, (r, j, n), 2)).astype(jnp.float32)
    return oh.sum(axis=1) * (1.0 / j)


def _gated_rows(table, gate_col):
    ms = jnp.mean(table * table, axis=1, keepdims=True)
    return table * lax.rsqrt(ms + 1e-6) * jax.nn.sigmoid(gate_col)


def _precompute_body(step_ref, note_table_ref, note_gate_ref,
                     transpose_table_ref, transpose_gate_ref, cmd_table_ref,
                     val_table_ref, dummy_vec_ref, fx_proj_ref,
                     table_proj_ref, gtt_ref, groove_proj_ref,
                     softsynths_ref, waveframes_ref, soft_proj_ref,
                     wave_proj_ref, instr_proj_ref, cp_ref, grooves_ref,
                     traces_c_ref, traces_v_ref, tables_c_ref, tables_v_ref,
                     instruments_ref, out_ref, idx_ref):
    fx_proj = fx_proj_ref[...]
    fx_a = fx_proj[:, :128]
    fx_b = fx_proj[:, 128:]
    cmd_table = cmd_table_ref[...]
    val_table128 = val_table_ref[...][:128, :]
    dummy = dummy_vec_ref[...]
    table_proj = table_proj_ref[...]
    ta = _mm(table_proj, fx_a)
    tb = _mm(table_proj, fx_b)

    ne_tab = _gated_rows(note_table_ref[...], note_gate_ref[...])
    te_tab = _gated_rows(transpose_table_ref[...],
                         transpose_gate_ref[...])[:128, :]

    cg = _count_mat(grooves_ref[...], 32)
    groove_emb = _mmT(_mm(cg, gtt_ref[...]), groove_proj_ref[...])

    def content_stage(c_arr, v_arr, table_branch_rows):
        c = c_arr % 16
        cmdbar = _mm(_count_mat(c, 16), cmd_table)
        m1 = c == 1
        m2 = c == 2
        m0 = jnp.logical_not(jnp.logical_or(m1, m2))
        vebar = (_mm(_count_mat(v_arr, 128, m2), groove_emb) +
                 _mm(_count_mat(v_arr, 128, m0), val_table128))
        if table_branch_rows is None:
            w1 = jnp.mean(m1.astype(jnp.float32), axis=1, keepdims=True)
            vebar = vebar + _mm(w1, dummy)
        else:
            vebar = vebar + _mm(_count_mat(v_arr, 128, m1), table_branch_rows)
        return _mmT(cmdbar, ta) + _mmT(vebar, tb)

    trace_emb = content_stage(traces_c_ref[...], traces_v_ref[...], None)
    full_tab = content_stage(tables_c_ref[...], tables_v_ref[...], trace_emb)

    instr = instruments_ref[...]
    s_e = _mm(_onehot(instr[:, 0:1] % 128, 128), softsynths_ref[...])
    w_e = _mm(_onehot(instr[:, 1:2] % 128, 128), waveframes_ref[...])
    t_e = _mm(_onehot(instr[:, 2:3] % 128, 128), full_tab)
    p = (instr[:, 3:8] % 16).astype(jnp.float32) * (1.0 / 16.0)
    instr_proj = instr_proj_ref[...]
    instr_emb = (_mmT(p, instr_proj[:, 0:5]) +
                 _mmT(_mmT(s_e, soft_proj_ref[...]), instr_proj[:, 5:69]) +
                 _mmT(_mmT(w_e, wave_proj_ref[...]), instr_proj[:, 69:101]) +
                 _mmT(t_e, instr_proj[:, 101:165]))

    for c in range(4):
        cpc = cp_ref[c]
        cp_n = cpc[:, 0:128]
        cp_i = cpc[:, 128:256]
        cp_f = cpc[:, 256:384]
        cp_t = cpc[:, 384:400]
        p_c = _mm(cp_f, fx_a)
        n_c = _mm(cp_f, fx_b)
        dve_generic = _mmT(val_table128, n_c)
        out_ref[_A_OFF + c * 128:_A_OFF + (c + 1) * 128, :] = _mmT(ne_tab, cp_n)
        out_ref[_B_OFF + c * 128:_B_OFF + (c + 1) * 128, :] = _mmT(instr_emb, cp_i)
        out_ref[_DCMD_OFF + c * 16:_DCMD_OFF + (c + 1) * 16, :] = _mmT(cmd_table, p_c)
        base = _DVE_OFF + c * 384
        out_ref[base:base + 128, :] = dve_generic
        out_ref[base + 128:base + 256, :] = _mmT(full_tab, n_c)
        out_ref[base + 256:base + 384, :] = _mmT(groove_emb, n_c)
        out_ref[_VT_OFF + c * 128:_VT_OFF + (c + 1) * 128, :] = dve_generic * (1.0 / 16.0)
        out_ref[_E_OFF + c * 128:_E_OFF + (c + 1) * 128, :] = _mmT(te_tab, cp_t)
    out_ref[_ZERO_ROW:_N_ROWS, :] = jnp.zeros((_N_ROWS - _ZERO_ROW, _D),
                                              jnp.float32)

    st = step_ref[...]
    cvec = lax.broadcasted_iota(jnp.int32, (_NW * _B_PER_W, 1), 0) & 3
    c2 = st[:, 2:3] & 15
    sel = jnp.where(c2 == 1, 1, jnp.where(c2 == 2, 2, 0))
    idx_ref[...] = jnp.concatenate([
        _A_OFF + cvec * 128 + (st[:, 0:1] & 127),
        _B_OFF + cvec * 128 + (st[:, 1:2] & 127),
        _DCMD_OFF + cvec * 16 + c2,
        _DVE_OFF + cvec * 384 + sel * 128 + (st[:, 3:4] & 127),
        _VT_OFF + cvec * 128 + (st[:, 4:20] & 127),
        _E_OFF + cvec * 128 + (st[:, 20:21] & 127),
        jnp.full((_NW * _B_PER_W, _KP - _K), _ZERO_ROW, jnp.int32),
    ], axis=1)


def _precompute_table(step, note_table, note_gate, transpose_table,
                      transpose_gate, cmd_table, val_table, dummy_vec,
                      fx_proj, table_proj, groove_tick_table, groove_proj,
                      softsynths, waveframes, soft_proj, wave_proj,
                      instr_proj, channel_projections, grooves, traces,
                      tables, instruments):
    return pl.pallas_call(
        _precompute_body,
        out_shape=(jax.ShapeDtypeStruct((_N_ROWS, _D), jnp.float32),
                   jax.ShapeDtypeStruct((_NW * _B_PER_W, _KP), jnp.int32)),
    )(step, note_table, note_gate.reshape(128, 1), transpose_table,
      transpose_gate.reshape(256, 1), cmd_table, val_table,
      dummy_vec.reshape(1, 64), fx_proj, table_proj, groove_tick_table,
      groove_proj, softsynths, waveframes, soft_proj, wave_proj, instr_proj,
      channel_projections, grooves, traces[:, :, 0], traces[:, :, 1],
      tables[:, :, 0], tables[:, :, 1], instruments)


def _sc_gather_sum(table, ids):
    mesh = plsc.VectorSubcoreMesh(core_axis_name="c", subcore_axis_name="s")

    @functools.partial(
        pl.kernel,
        mesh=mesh,
        out_type=jax.ShapeDtypeStruct((_NW * _B_PER_W, _D), jnp.float32),
        scratch_types=[
            pltpu.VMEM((_B_PER_W, _KP), jnp.int32),
            pltpu.VMEM((_B_PER_W, _K, _D), jnp.float32),
            pltpu.VMEM((_B_PER_W, _D), jnp.float32),
            pltpu.SemaphoreType.DMA,
        ],
    )
    def k(table_hbm, ids_hbm, out_hbm, idx_v, buf_v, acc_v, sem):
        wid = lax.axis_index("s") * 2 + lax.axis_index("c")
        pltpu.sync_copy(ids_hbm.at[pl.ds(wid * _B_PER_W, _B_PER_W)], idx_v)
        copies = [
            pltpu.async_copy(table_hbm.at[idx_v.at[o, pl.ds(0, _K)]],
                             buf_v.at[o], sem)
            for o in range(_B_PER_W)
        ]
        for c in copies:
            c.wait()

        def body(o, carry):
            for v in range(_D // 16):
                sl = pl.ds(v * 16, 16)
                acc = buf_v[o, 0, sl]
                for j in range(1, _K):
                    acc = acc + buf_v[o, j, sl]
                acc_v[o, sl] = acc
            return carry

        lax.fori_loop(0, _B_PER_W, body, 0)
        pltpu.sync_copy(acc_v, out_hbm.at[pl.ds(wid * _B_PER_W, _B_PER_W)])

    return k(table, ids)


def kernel(step, note_table, note_gate, transpose_table, transpose_gate,
           cmd_table, val_table, dummy_vec, fx_proj, table_proj,
           groove_tick_table, groove_proj, softsynths, waveframes, soft_proj,
           wave_proj, instr_proj, channel_projections, grooves, traces,
           tables, instruments):
    tab, ids = _precompute_table(
        step.astype(jnp.int32).reshape(_NW * _B_PER_W, _K), note_table,
        note_gate, transpose_table, transpose_gate, cmd_table, val_table,
        dummy_vec, fx_proj, table_proj, groove_tick_table, groove_proj,
        softsynths, waveframes, soft_proj, wave_proj, instr_proj,
        channel_projections, grooves, traces, tables, instruments)
    out = _sc_gather_sum(tab, ids)
    return out.reshape(256, 4, _D)


# speedup vs baseline: 1.3168x; 1.3168x over previous
import functools

import jax
import jax.numpy as jnp
from jax import lax
from jax.experimental import pallas as pl
from jax.experimental.pallas import tpu as pltpu
from jax.experimental.pallas import tpu_sc as plsc

_A_OFF = 0
_B_OFF = 512
_DCMD_OFF = 1024
_DVE_OFF = 1088
_VT_OFF = 2624
_E_OFF = 3136
_ZERO_ROW = 3648
_N_ROWS = 3656

_NW = 32
_B_PER_W = 32
_K = 21
_KP = 24
_D = 128

_WSPEC = [("note_table", 128), ("note_gate", 128), ("transpose_table", 256),
          ("transpose_gate", 256), ("cmd_table", 16), ("val_table", 256),
          ("dummy", 8), ("fxa", 128), ("fxb", 128), ("table_proj", 64),
          ("gtt", 32), ("groove_proj", 64), ("softsynths", 128),
          ("waveframes", 128), ("soft_proj", 64), ("wave_proj", 32),
          ("ip1", 128), ("ip2", 128), ("cpn", 512), ("cpi", 512),
          ("cpf", 512), ("cpt", 512)]
_ISPEC = [("grooves", 128), ("traces_c", 128), ("traces_v", 128),
          ("tables_c", 128), ("tables_v", 128), ("instruments", 128)]


def _offsets(spec):
    out, acc = {}, 0
    for name, rows in spec:
        out[name] = (acc, rows)
        acc += rows
    return out, acc


_WOFF, _W_ROWS = _offsets(_WSPEC)
_IOFF, _I_ROWS = _offsets(_ISPEC)


def _mm(x, y):
    return lax.dot_general(x, y, (((1,), (0,)), ((), ())),
                           precision=lax.Precision.HIGHEST,
                           preferred_element_type=jnp.float32)


def _mmT(x, y):
    return lax.dot_general(x, y, (((1,), (1,)), ((), ())),
                           precision=lax.Precision.HIGHEST,
                           preferred_element_type=jnp.float32)


def _onehot(idx_col, n):
    iota = lax.broadcasted_iota(jnp.int32, (idx_col.shape[0], n), 1)
    return (idx_col == iota).astype(jnp.float32)


def _count_mat(idx2, n, mask=None):
    r, j = idx2.shape
    if mask is not None:
        idx2 = jnp.where(mask, idx2, n)
    oh = (idx2[:, :, None] ==
          lax.broadcasted_iota(jnp.int32, (r, j, n), 2)).astype(jnp.float32)
    return oh.sum(axis=1) * (1.0 / j)


def _gated_rows(table, gate_col):
    ms = jnp.mean(table * table, axis=1, keepdims=True)
    return table * lax.rsqrt(ms + 1e-6) * jax.nn.sigmoid(gate_col)


def _precompute_body(step_ref, w_ref, i_ref, out_ref, idx_ref):
    def w(name, lanes=128):
        off, rows = _WOFF[name]
        return w_ref[off:off + rows, :lanes]

    def iw(name, lanes=128):
        off, rows = _IOFF[name]
        return i_ref[off:off + rows, :lanes]

    fx_a = w("fxa")
    fx_b = w("fxb", 64)
    cmd_table = w("cmd_table")
    val_table128 = w("val_table", 64)[:128, :]
    dummy = w("dummy", 64)[:1, :]
    table_proj = w("table_proj")
    ta = _mm(table_proj, fx_a)
    tb = _mm(table_proj, fx_b)

    ne_tab = _gated_rows(w("note_table"), w("note_gate", 1))
    te_tab = _gated_rows(w("transpose_table", 16),
                         w("transpose_gate", 1))[:128, :]

    cg = _count_mat(iw("grooves", 16), 32)
    groove_emb = _mmT(_mm(cg, w("gtt", 64)), w("groove_proj", 64))

    def content_stage(c_arr, v_arr, table_branch_rows):
        c = c_arr % 16
        cmdbar = _mm(_count_mat(c, 16), cmd_table)
        m1 = c == 1
        m2 = c == 2
        m0 = jnp.logical_not(jnp.logical_or(m1, m2))
        vebar = (_mm(_count_mat(v_arr, 128, m2), groove_emb) +
                 _mm(_count_mat(v_arr, 128, m0), val_table128))
        if table_branch_rows is None:
            w1 = jnp.mean(m1.astype(jnp.float32), axis=1, keepdims=True)
            vebar = vebar + _mm(w1, dummy)
        else:
            vebar = vebar + _mm(_count_mat(v_arr, 128, m1), table_branch_rows)
        return _mmT(cmdbar, ta) + _mmT(vebar, tb)

    trace_emb = content_stage(iw("traces_c", 16), iw("traces_v", 16), None)
    full_tab = content_stage(iw("tables_c", 16), iw("tables_v", 16), trace_emb)

    instr = iw("instruments", 8)
    s_e = _mm(_onehot(instr[:, 0:1] % 128, 128), w("softsynths", 16))
    w_e = _mm(_onehot(instr[:, 1:2] % 128, 128), w("waveframes", 32))
    t_e = _mm(_onehot(instr[:, 2:3] % 128, 128), full_tab)
    p = (instr[:, 3:8] % 16).astype(jnp.float32) * (1.0 / 16.0)
    ip1 = w("ip1", 101)
    instr_emb = (_mmT(p, ip1[:, 0:5]) +
                 _mmT(_mmT(s_e, w("soft_proj", 16)), ip1[:, 5:69]) +
                 _mmT(_mmT(w_e, w("wave_proj", 32)), ip1[:, 69:101]) +
                 _mmT(t_e, w("ip2", 64)))

    for c in range(4):
        cp_n = w("cpn")[c * 128:(c + 1) * 128, :]
        cp_i = w("cpi")[c * 128:(c + 1) * 128, :]
        cp_f = w("cpf")[c * 128:(c + 1) * 128, :]
        cp_t = w("cpt", 16)[c * 128:(c + 1) * 128, :]
        p_c = _mm(cp_f, fx_a)
        n_c = _mm(cp_f, fx_b)
        dve_generic = _mmT(val_table128, n_c)
        out_ref[_A_OFF + c * 128:_A_OFF + (c + 1) * 128, :] = _mmT(ne_tab, cp_n)
        out_ref[_B_OFF + c * 128:_B_OFF + (c + 1) * 128, :] = _mmT(instr_emb, cp_i)
        out_ref[_DCMD_OFF + c * 16:_DCMD_OFF + (c + 1) * 16, :] = _mmT(cmd_table, p_c)
        base = _DVE_OFF + c * 384
        out_ref[base:base + 128, :] = dve_generic
        out_ref[base + 128:base + 256, :] = _mmT(full_tab, n_c)
        out_ref[base + 256:base + 384, :] = _mmT(groove_emb, n_c)
        out_ref[_VT_OFF + c * 128:_VT_OFF + (c + 1) * 128, :] = dve_generic * (1.0 / 16.0)
        out_ref[_E_OFF + c * 128:_E_OFF + (c + 1) * 128, :] = _mmT(te_tab, cp_t)
    out_ref[_ZERO_ROW:_N_ROWS, :] = jnp.zeros((_N_ROWS - _ZERO_ROW, _D),
                                              jnp.float32)

    st = step_ref[...]
    cvec = lax.broadcasted_iota(jnp.int32, (_NW * _B_PER_W, 1), 0) & 3
    c2 = st[:, 2:3] & 15
    sel = jnp.where(c2 == 1, 1, jnp.where(c2 == 2, 2, 0))
    idx_ref[...] = jnp.concatenate([
        _A_OFF + cvec * 128 + (st[:, 0:1] & 127),
        _B_OFF + cvec * 128 + (st[:, 1:2] & 127),
        _DCMD_OFF + cvec * 16 + c2,
        _DVE_OFF + cvec * 384 + sel * 128 + (st[:, 3:4] & 127),
        _VT_OFF + cvec * 128 + (st[:, 4:20] & 127),
        _E_OFF + cvec * 128 + (st[:, 20:21] & 127),
        jnp.full((_NW * _B_PER_W, _KP - _K), _ZERO_ROW, jnp.int32),
    ], axis=1)


def _pack(spec, parts):
    blocks = []
    for name, rows in spec:
        x = parts[name]
        r, c = x.shape
        assert r <= rows and c <= 128, (name, x.shape)
        blocks.append(jnp.pad(x, ((0, rows - r), (0, 128 - c))))
    return jnp.concatenate(blocks, axis=0)


def _precompute_table(step, note_table, note_gate, transpose_table,
                      transpose_gate, cmd_table, val_table, dummy_vec,
                      fx_proj, table_proj, groove_tick_table, groove_proj,
                      softsynths, waveframes, soft_proj, wave_proj,
                      instr_proj, channel_projections, grooves, traces,
                      tables, instruments):
    cp = channel_projections.reshape(512, 400)
    wbuf = _pack(_WSPEC, {
        "note_table": note_table, "note_gate": note_gate.reshape(128, 1),
        "transpose_table": transpose_table,
        "transpose_gate": transpose_gate.reshape(256, 1),
        "cmd_table": cmd_table, "val_table": val_table,
        "dummy": dummy_vec.reshape(1, 64), "fxa": fx_proj[:, :128],
        "fxb": fx_proj[:, 128:], "table_proj": table_proj,
        "gtt": groove_tick_table, "groove_proj": groove_proj,
        "softsynths": softsynths, "waveframes": waveframes,
        "soft_proj": soft_proj, "wave_proj": wave_proj,
        "ip1": instr_proj[:, 0:101], "ip2": instr_proj[:, 101:165],
        "cpn": cp[:, 0:128], "cpi": cp[:, 128:256], "cpf": cp[:, 256:384],
        "cpt": cp[:, 384:400],
    })
    ibuf = _pack(_ISPEC, {
        "grooves": grooves, "traces_c": traces[:, :, 0],
        "traces_v": traces[:, :, 1], "tables_c": tables[:, :, 0],
        "tables_v": tables[:, :, 1], "instruments": instruments,
    })
    return pl.pallas_call(
        _precompute_body,
        out_shape=(jax.ShapeDtypeStruct((_N_ROWS, _D), jnp.float32),
                   jax.ShapeDtypeStruct((_NW * _B_PER_W, _KP), jnp.int32)),
    )(step, wbuf, ibuf)


def _sc_gather_sum(table, ids):
    mesh = plsc.VectorSubcoreMesh(core_axis_name="c", subcore_axis_name="s")

    @functools.partial(
        pl.kernel,
        mesh=mesh,
        out_type=jax.ShapeDtypeStruct((_NW * _B_PER_W, _D), jnp.float32),
        scratch_types=[
            pltpu.VMEM((_B_PER_W, _KP), jnp.int32),
            pltpu.VMEM((_B_PER_W, _K, _D), jnp.float32),
            pltpu.VMEM((_B_PER_W, _D), jnp.float32),
            pltpu.SemaphoreType.DMA,
        ],
    )
    def k(table_hbm, ids_hbm, out_hbm, idx_v, buf_v, acc_v, sem):
        wid = lax.axis_index("s") * 2 + lax.axis_index("c")
        pltpu.sync_copy(ids_hbm.at[pl.ds(wid * _B_PER_W, _B_PER_W)], idx_v)
        copies = [
            pltpu.async_copy(table_hbm.at[idx_v.at[o, pl.ds(0, _K)]],
                             buf_v.at[o], sem)
            for o in range(_B_PER_W)
        ]
        for c in copies:
            c.wait()

        def body(o, carry):
            for v in range(_D // 16):
                sl = pl.ds(v * 16, 16)
                acc = buf_v[o, 0, sl]
                for j in range(1, _K):
                    acc = acc + buf_v[o, j, sl]
                acc_v[o, sl] = acc
            return carry

        lax.fori_loop(0, _B_PER_W, body, 0)
        pltpu.sync_copy(acc_v, out_hbm.at[pl.ds(wid * _B_PER_W, _B_PER_W)])

    return k(table, ids)


def kernel(step, note_table, note_gate, transpose_table, transpose_gate,
           cmd_table, val_table, dummy_vec, fx_proj, table_proj,
           groove_tick_table, groove_proj, softsynths, waveframes, soft_proj,
           wave_proj, instr_proj, channel_projections, grooves, traces,
           tables, instruments):
    tab, ids = _precompute_table(
        step.astype(jnp.int32).reshape(_NW * _B_PER_W, _K), note_table,
        note_gate, transpose_table, transpose_gate, cmd_table, val_table,
        dummy_vec, fx_proj, table_proj, groove_tick_table, groove_proj,
        softsynths, waveframes, soft_proj, wave_proj, instr_proj,
        channel_projections, grooves, traces, tables, instruments)
    out = _sc_gather_sum(tab, ids)
    return out.reshape(256, 4, _D)


# speedup vs baseline: 1.5546x; 1.1805x over previous
import functools

import jax
import jax.numpy as jnp
from jax import lax
from jax.experimental import pallas as pl
from jax.experimental.pallas import tpu as pltpu
from jax.experimental.pallas import tpu_sc as plsc

_A_OFF = 0
_B_OFF = 512
_DCMD_OFF = 1024
_DVE_OFF = 1088
_E_OFF = 2624
_ZERO_ROW = 3136
_N_ROWS = 3144

_NW = 32
_B_PER_W = 32
_K = 5
_KP = 8
_D = 128

_WSPEC = [("note_table", 128), ("note_gate", 128), ("transpose_table", 256),
          ("transpose_gate", 256), ("cmd_table", 16), ("val_table", 256),
          ("dummy", 8), ("fxa", 128), ("fxb", 128), ("table_proj", 64),
          ("gtt", 32), ("groove_proj", 64), ("softsynths", 128),
          ("waveframes", 128), ("soft_proj", 64), ("wave_proj", 32),
          ("ip1", 128), ("ip2", 128), ("cpn", 512), ("cpi", 512),
          ("cpf", 512), ("cpt", 512)]
_ISPEC = [("grooves", 128), ("traces_c", 128), ("traces_v", 128),
          ("tables_c", 128), ("tables_v", 128), ("instruments", 128)]


def _offsets(spec):
    out, acc = {}, 0
    for name, rows in spec:
        out[name] = (acc, rows)
        acc += rows
    return out, acc


_WOFF, _W_ROWS = _offsets(_WSPEC)
_IOFF, _I_ROWS = _offsets(_ISPEC)


def _mm(x, y):
    return lax.dot_general(x, y, (((1,), (0,)), ((), ())),
                           precision=lax.Precision.HIGHEST,
                           preferred_element_type=jnp.float32)


def _mmT(x, y):
    return lax.dot_general(x, y, (((1,), (1,)), ((), ())),
                           precision=lax.Precision.HIGHEST,
                           preferred_element_type=jnp.float32)


def _onehot(idx_col, n):
    iota = lax.broadcasted_iota(jnp.int32, (idx_col.shape[0], n), 1)
    return (idx_col == iota).astype(jnp.float32)


def _count_mat(idx2, n, mask=None):
    r, j = idx2.shape
    if mask is not None:
        idx2 = jnp.where(mask, idx2, n)
    oh = (idx2[:, :, None] ==
          lax.broadcasted_iota(jnp.int32, (r, j, n), 2)).astype(jnp.float32)
    return oh.sum(axis=1) * (1.0 / j)


def _gated_rows(table, gate_col):
    ms = jnp.mean(table * table, axis=1, keepdims=True)
    return table * lax.rsqrt(ms + 1e-6) * jax.nn.sigmoid(gate_col)


def _precompute_body(step_ref, w_ref, i_ref, out_ref, idx_ref, aux_ref):
    def w(name, lanes=128):
        off, rows = _WOFF[name]
        return w_ref[off:off + rows, :lanes]

    def iw(name, lanes=128):
        off, rows = _IOFF[name]
        return i_ref[off:off + rows, :lanes]

    fx_a = w("fxa")
    fx_b = w("fxb", 64)
    cmd_table = w("cmd_table")
    val_table128 = w("val_table", 64)[:128, :]
    dummy = w("dummy", 64)[:1, :]
    table_proj = w("table_proj")
    ta = _mm(table_proj, fx_a)
    tb = _mm(table_proj, fx_b)

    ne_tab = _gated_rows(w("note_table"), w("note_gate", 1))
    te_tab = _gated_rows(w("transpose_table", 16),
                         w("transpose_gate", 1))[:128, :]

    cg = _count_mat(iw("grooves", 16), 32)
    groove_emb = _mmT(_mm(cg, w("gtt", 64)), w("groove_proj", 64))

    def content_stage(c_arr, v_arr, table_branch_rows):
        c = c_arr % 16
        cmdbar = _mm(_count_mat(c, 16), cmd_table)
        m1 = c == 1
        m2 = c == 2
        m0 = jnp.logical_not(jnp.logical_or(m1, m2))
        vebar = (_mm(_count_mat(v_arr, 128, m2), groove_emb) +
                 _mm(_count_mat(v_arr, 128, m0), val_table128))
        if table_branch_rows is None:
            w1 = jnp.mean(m1.astype(jnp.float32), axis=1, keepdims=True)
            vebar = vebar + _mm(w1, dummy)
        else:
            vebar = vebar + _mm(_count_mat(v_arr, 128, m1), table_branch_rows)
        return _mmT(cmdbar, ta) + _mmT(vebar, tb)

    trace_emb = content_stage(iw("traces_c", 16), iw("traces_v", 16), None)
    full_tab = content_stage(iw("tables_c", 16), iw("tables_v", 16), trace_emb)

    instr = iw("instruments", 8)
    s_e = _mm(_onehot(instr[:, 0:1] % 128, 128), w("softsynths", 16))
    w_e = _mm(_onehot(instr[:, 1:2] % 128, 128), w("waveframes", 32))
    t_e = _mm(_onehot(instr[:, 2:3] % 128, 128), full_tab)
    p = (instr[:, 3:8] % 16).astype(jnp.float32) * (1.0 / 16.0)
    ip1 = w("ip1", 101)
    instr_emb = (_mmT(p, ip1[:, 0:5]) +
                 _mmT(_mmT(s_e, w("soft_proj", 16)), ip1[:, 5:69]) +
                 _mmT(_mmT(w_e, w("wave_proj", 32)), ip1[:, 69:101]) +
                 _mmT(t_e, w("ip2", 64)))

    ncs = []
    for c in range(4):
        cp_n = w("cpn")[c * 128:(c + 1) * 128, :]
        cp_i = w("cpi")[c * 128:(c + 1) * 128, :]
        cp_f = w("cpf")[c * 128:(c + 1) * 128, :]
        cp_t = w("cpt", 16)[c * 128:(c + 1) * 128, :]
        p_c = _mm(cp_f, fx_a)
        n_c = _mm(cp_f, fx_b)
        ncs.append(n_c)
        out_ref[_A_OFF + c * 128:_A_OFF + (c + 1) * 128, :] = _mmT(ne_tab, cp_n)
        out_ref[_B_OFF + c * 128:_B_OFF + (c + 1) * 128, :] = _mmT(instr_emb, cp_i)
        out_ref[_DCMD_OFF + c * 16:_DCMD_OFF + (c + 1) * 16, :] = _mmT(cmd_table, p_c)
        base = _DVE_OFF + c * 384
        out_ref[base:base + 128, :] = _mmT(val_table128, n_c)
        out_ref[base + 128:base + 256, :] = _mmT(full_tab, n_c)
        out_ref[base + 256:base + 384, :] = _mmT(groove_emb, n_c)
        out_ref[_E_OFF + c * 128:_E_OFF + (c + 1) * 128, :] = _mmT(te_tab, cp_t)
    out_ref[_ZERO_ROW:_N_ROWS, :] = jnp.zeros((_N_ROWS - _ZERO_ROW, _D),
                                              jnp.float32)

    st = step_ref[...]
    cvec = lax.broadcasted_iota(jnp.int32, (_NW * _B_PER_W, 1), 0) & 3
    c2 = st[:, 2:3] & 15
    sel = jnp.where(c2 == 1, 1, jnp.where(c2 == 2, 2, 0))
    idx_ref[...] = jnp.concatenate([
        _A_OFF + cvec * 128 + (st[:, 0:1] & 127),
        _B_OFF + cvec * 128 + (st[:, 1:2] & 127),
        _DCMD_OFF + cvec * 16 + c2,
        _DVE_OFF + cvec * 384 + sel * 128 + (st[:, 3:4] & 127),
        _E_OFF + cvec * 128 + (st[:, 20:21] & 127),
        jnp.full((_NW * _B_PER_W, _KP - _K), _ZERO_ROW, jnp.int32),
    ], axis=1)

    counts = _count_mat(st[:, 4:20] & 127, 128)
    vbar = _mm(counts, val_table128)
    aux = None
    for c in range(4):
        part = _mmT(vbar * (cvec == c).astype(jnp.float32), ncs[c])
        aux = part if aux is None else aux + part
    aux_ref[...] = aux


def _pack(spec, parts):
    blocks = []
    for name, rows in spec:
        x = parts[name]
        r, c = x.shape
        assert r <= rows and c <= 128, (name, x.shape)
        blocks.append(jnp.pad(x, ((0, rows - r), (0, 128 - c))))
    return jnp.concatenate(blocks, axis=0)


def _precompute_table(step, note_table, note_gate, transpose_table,
                      transpose_gate, cmd_table, val_table, dummy_vec,
                      fx_proj, table_proj, groove_tick_table, groove_proj,
                      softsynths, waveframes, soft_proj, wave_proj,
                      instr_proj, channel_projections, grooves, traces,
                      tables, instruments):
    cp = channel_projections.reshape(512, 400)
    wbuf = _pack(_WSPEC, {
        "note_table": note_table, "note_gate": note_gate.reshape(128, 1),
        "transpose_table": transpose_table,
        "transpose_gate": transpose_gate.reshape(256, 1),
        "cmd_table": cmd_table, "val_table": val_table,
        "dummy": dummy_vec.reshape(1, 64), "fxa": fx_proj[:, :128],
        "fxb": fx_proj[:, 128:], "table_proj": table_proj,
        "gtt": groove_tick_table, "groove_proj": groove_proj,
        "softsynths": softsynths, "waveframes": waveframes,
        "soft_proj": soft_proj, "wave_proj": wave_proj,
        "ip1": instr_proj[:, 0:101], "ip2": instr_proj[:, 101:165],
        "cpn": cp[:, 0:128], "cpi": cp[:, 128:256], "cpf": cp[:, 256:384],
        "cpt": cp[:, 384:400],
    })
    ibuf = _pack(_ISPEC, {
        "grooves": grooves, "traces_c": traces[:, :, 0],
        "traces_v": traces[:, :, 1], "tables_c": tables[:, :, 0],
        "tables_v": tables[:, :, 1], "instruments": instruments,
    })
    return pl.pallas_call(
        _precompute_body,
        out_shape=(jax.ShapeDtypeStruct((_N_ROWS, _D), jnp.float32),
                   jax.ShapeDtypeStruct((_NW * _B_PER_W, _KP), jnp.int32),
                   jax.ShapeDtypeStruct((_NW * _B_PER_W, _D), jnp.float32)),
    )(step, wbuf, ibuf)


def _sc_gather_sum(table, ids, aux):
    mesh = plsc.VectorSubcoreMesh(core_axis_name="c", subcore_axis_name="s")

    @functools.partial(
        pl.kernel,
        mesh=mesh,
        out_type=jax.ShapeDtypeStruct((_NW * _B_PER_W, _D), jnp.float32),
        scratch_types=[
            pltpu.VMEM((_B_PER_W, _KP), jnp.int32),
            pltpu.VMEM((_B_PER_W, _K, _D), jnp.float32),
            pltpu.VMEM((_B_PER_W, _D), jnp.float32),
            pltpu.SemaphoreType.DMA,
            pltpu.SemaphoreType.DMA,
        ],
    )
    def k(table_hbm, ids_hbm, aux_hbm, out_hbm, idx_v, buf_v, acc_v, sem,
          sem2):
        wid = lax.axis_index("s") * 2 + lax.axis_index("c")
        row0 = wid * _B_PER_W
        pltpu.sync_copy(ids_hbm.at[pl.ds(row0, _B_PER_W)], idx_v)
        aux_cp = pltpu.async_copy(aux_hbm.at[pl.ds(row0, _B_PER_W)], acc_v,
                                  sem2)
        copies = [
            pltpu.async_copy(table_hbm.at[idx_v.at[o, pl.ds(0, _K)]],
                             buf_v.at[o], sem)
            for o in range(_B_PER_W)
        ]
        aux_cp.wait()
        for o in range(_B_PER_W):
            copies[o].wait()
            for v in range(_D // 16):
                sl = pl.ds(v * 16, 16)
                acc = acc_v[o, sl]
                for j in range(_K):
                    acc = acc + buf_v[o, j, sl]
                acc_v[o, sl] = acc
        pltpu.sync_copy(acc_v, out_hbm.at[pl.ds(row0, _B_PER_W)])

    return k(table, ids, aux)


def kernel(step, note_table, note_gate, transpose_table, transpose_gate,
           cmd_table, val_table, dummy_vec, fx_proj, table_proj,
           groove_tick_table, groove_proj, softsynths, waveframes, soft_proj,
           wave_proj, instr_proj, channel_projections, grooves, traces,
           tables, instruments):
    tab, ids, aux = _precompute_table(
        step.astype(jnp.int32).reshape(_NW * _B_PER_W, 21), note_table,
        note_gate, transpose_table, transpose_gate, cmd_table, val_table,
        dummy_vec, fx_proj, table_proj, groove_tick_table, groove_proj,
        softsynths, waveframes, soft_proj, wave_proj, instr_proj,
        channel_projections, grooves, traces, tables, instruments)
    out = _sc_gather_sum(tab, ids, aux)
    return out.reshape(256, 4, _D)


# speedup vs baseline: 1.7436x; 1.1216x over previous
import functools

import jax
import jax.numpy as jnp
from jax import lax
from jax.experimental import pallas as pl
from jax.experimental.pallas import tpu as pltpu
from jax.experimental.pallas import tpu_sc as plsc

_A_OFF = 0
_B_OFF = 512
_DCMD_OFF = 1024
_DVE_OFF = 1088
_E_OFF = 2624
_ZERO_ROW = 3136
_N_ROWS = 3144

_NW = 32
_B_PER_W = 32
_K = 5
_KP = 8
_D = 128

_WSPEC = [("note_table", 128), ("note_gate", 128), ("transpose_table", 256),
          ("transpose_gate", 256), ("cmd_table", 16), ("val_table", 256),
          ("dummy", 8), ("fxa", 128), ("fxb", 128), ("table_proj", 64),
          ("gtt", 32), ("groove_proj", 64), ("softsynths", 128),
          ("waveframes", 128), ("soft_proj", 64), ("wave_proj", 32),
          ("ip1", 128), ("ip2", 128), ("cpn", 512), ("cpi", 512),
          ("cpf", 512), ("cpt", 512)]
_ISPEC = [("grooves", 128), ("traces_c", 128), ("traces_v", 128),
          ("tables_c", 128), ("tables_v", 128), ("instruments", 128)]


def _offsets(spec):
    out, acc = {}, 0
    for name, rows in spec:
        out[name] = (acc, rows)
        acc += rows
    return out, acc


_WOFF, _W_ROWS = _offsets(_WSPEC)
_IOFF, _I_ROWS = _offsets(_ISPEC)


def _mm(x, y):
    return lax.dot_general(x, y, (((1,), (0,)), ((), ())),
                           precision=lax.Precision.DEFAULT,
                           preferred_element_type=jnp.float32)


def _mmT(x, y):
    return lax.dot_general(x, y, (((1,), (1,)), ((), ())),
                           precision=lax.Precision.DEFAULT,
                           preferred_element_type=jnp.float32)


def _onehot(idx_col, n):
    iota = lax.broadcasted_iota(jnp.int32, (idx_col.shape[0], n), 1)
    return (idx_col == iota).astype(jnp.float32)


def _count_mat(idx2, n, mask=None):
    r, j = idx2.shape
    if mask is not None:
        idx2 = jnp.where(mask, idx2, n)
    oh = (idx2[:, :, None] ==
          lax.broadcasted_iota(jnp.int32, (r, j, n), 2)).astype(jnp.float32)
    return oh.sum(axis=1) * (1.0 / j)


def _gated_rows(table, gate_col):
    ms = jnp.mean(table * table, axis=1, keepdims=True)
    return table * lax.rsqrt(ms + 1e-6) * jax.nn.sigmoid(gate_col)


def _precompute_body(step_ref, w_ref, i_ref, out_ref, idx_ref, aux_ref):
    def w(name, lanes=128):
        off, rows = _WOFF[name]
        return w_ref[off:off + rows, :lanes]

    def iw(name, lanes=128):
        off, rows = _IOFF[name]
        return i_ref[off:off + rows, :lanes]

    fx_a = w("fxa")
    fx_b = w("fxb", 64)
    cmd_table = w("cmd_table")
    val_table128 = w("val_table", 64)[:128, :]
    dummy = w("dummy", 64)[:1, :]
    table_proj = w("table_proj")
    ta = _mm(table_proj, fx_a)
    tb = _mm(table_proj, fx_b)

    ne_tab = _gated_rows(w("note_table"), w("note_gate", 1))
    te_tab = _gated_rows(w("transpose_table", 16),
                         w("transpose_gate", 1))[:128, :]

    cg = _count_mat(iw("grooves", 16), 32)
    groove_emb = _mmT(_mm(cg, w("gtt", 64)), w("groove_proj", 64))

    def content_stage(c_arr, v_arr, table_branch_rows):
        c = c_arr % 16
        cmdbar = _mm(_count_mat(c, 16), cmd_table)
        m1 = c == 1
        m2 = c == 2
        m0 = jnp.logical_not(jnp.logical_or(m1, m2))
        vebar = (_mm(_count_mat(v_arr, 128, m2), groove_emb) +
                 _mm(_count_mat(v_arr, 128, m0), val_table128))
        if table_branch_rows is None:
            w1 = jnp.mean(m1.astype(jnp.float32), axis=1, keepdims=True)
            vebar = vebar + _mm(w1, dummy)
        else:
            vebar = vebar + _mm(_count_mat(v_arr, 128, m1), table_branch_rows)
        return _mmT(cmdbar, ta) + _mmT(vebar, tb)

    trace_emb = content_stage(iw("traces_c", 16), iw("traces_v", 16), None)
    full_tab = content_stage(iw("tables_c", 16), iw("tables_v", 16), trace_emb)

    instr = iw("instruments", 8)
    s_e = _mm(_onehot(instr[:, 0:1] % 128, 128), w("softsynths", 16))
    w_e = _mm(_onehot(instr[:, 1:2] % 128, 128), w("waveframes", 32))
    t_e = _mm(_onehot(instr[:, 2:3] % 128, 128), full_tab)
    p = (instr[:, 3:8] % 16).astype(jnp.float32) * (1.0 / 16.0)
    ip1 = w("ip1", 101)
    instr_emb = (_mmT(p, ip1[:, 0:5]) +
                 _mmT(_mmT(s_e, w("soft_proj", 16)), ip1[:, 5:69]) +
                 _mmT(_mmT(w_e, w("wave_proj", 32)), ip1[:, 69:101]) +
                 _mmT(t_e, w("ip2", 64)))

    ncs = []
    for c in range(4):
        cp_n = w("cpn")[c * 128:(c + 1) * 128, :]
        cp_i = w("cpi")[c * 128:(c + 1) * 128, :]
        cp_f = w("cpf")[c * 128:(c + 1) * 128, :]
        cp_t = w("cpt", 16)[c * 128:(c + 1) * 128, :]
        p_c = _mm(cp_f, fx_a)
        n_c = _mm(cp_f, fx_b)
        ncs.append(n_c)
        out_ref[_A_OFF + c * 128:_A_OFF + (c + 1) * 128, :] = _mmT(ne_tab, cp_n)
        out_ref[_B_OFF + c * 128:_B_OFF + (c + 1) * 128, :] = _mmT(instr_emb, cp_i)
        out_ref[_DCMD_OFF + c * 16:_DCMD_OFF + (c + 1) * 16, :] = _mmT(cmd_table, p_c)
        base = _DVE_OFF + c * 384
        out_ref[base:base + 128, :] = _mmT(val_table128, n_c)
        out_ref[base + 128:base + 256, :] = _mmT(full_tab, n_c)
        out_ref[base + 256:base + 384, :] = _mmT(groove_emb, n_c)
        out_ref[_E_OFF + c * 128:_E_OFF + (c + 1) * 128, :] = _mmT(te_tab, cp_t)
    out_ref[_ZERO_ROW:_N_ROWS, :] = jnp.zeros((_N_ROWS - _ZERO_ROW, _D),
                                              jnp.float32)

    st = step_ref[...]
    cvec = lax.broadcasted_iota(jnp.int32, (_NW * _B_PER_W, 1), 0) & 3
    c2 = st[:, 2:3] & 15
    sel = jnp.where(c2 == 1, 1, jnp.where(c2 == 2, 2, 0))
    idx_ref[...] = jnp.concatenate([
        _A_OFF + cvec * 128 + (st[:, 0:1] & 127),
        _B_OFF + cvec * 128 + (st[:, 1:2] & 127),
        _DCMD_OFF + cvec * 16 + c2,
        _DVE_OFF + cvec * 384 + sel * 128 + (st[:, 3:4] & 127),
        _E_OFF + cvec * 128 + (st[:, 20:21] & 127),
        jnp.full((_NW * _B_PER_W, _KP - _K), _ZERO_ROW, jnp.int32),
    ], axis=1)

    counts = _count_mat(st[:, 4:20] & 127, 128)
    vbar = _mm(counts, val_table128)
    aux = None
    for c in range(4):
        part = _mmT(vbar * (cvec == c).astype(jnp.float32), ncs[c])
        aux = part if aux is None else aux + part
    aux_ref[...] = aux


def _pack(spec, parts):
    blocks = []
    for name, rows in spec:
        x = parts[name]
        r, c = x.shape
        assert r <= rows and c <= 128, (name, x.shape)
        blocks.append(jnp.pad(x, ((0, rows - r), (0, 128 - c))))
    return jnp.concatenate(blocks, axis=0)


def _precompute_table(step, note_table, note_gate, transpose_table,
                      transpose_gate, cmd_table, val_table, dummy_vec,
                      fx_proj, table_proj, groove_tick_table, groove_proj,
                      softsynths, waveframes, soft_proj, wave_proj,
                      instr_proj, channel_projections, grooves, traces,
                      tables, instruments):
    cp = channel_projections.reshape(512, 400)
    wbuf = _pack(_WSPEC, {
        "note_table": note_table, "note_gate": note_gate.reshape(128, 1),
        "transpose_table": transpose_table,
        "transpose_gate": transpose_gate.reshape(256, 1),
        "cmd_table": cmd_table, "val_table": val_table,
        "dummy": dummy_vec.reshape(1, 64), "fxa": fx_proj[:, :128],
        "fxb": fx_proj[:, 128:], "table_proj": table_proj,
        "gtt": groove_tick_table, "groove_proj": groove_proj,
        "softsynths": softsynths, "waveframes": waveframes,
        "soft_proj": soft_proj, "wave_proj": wave_proj,
        "ip1": instr_proj[:, 0:101], "ip2": instr_proj[:, 101:165],
        "cpn": cp[:, 0:128], "cpi": cp[:, 128:256], "cpf": cp[:, 256:384],
        "cpt": cp[:, 384:400],
    })
    ibuf = _pack(_ISPEC, {
        "grooves": grooves, "traces_c": traces[:, :, 0],
        "traces_v": traces[:, :, 1], "tables_c": tables[:, :, 0],
        "tables_v": tables[:, :, 1], "instruments": instruments,
    })
    return pl.pallas_call(
        _precompute_body,
        out_shape=(jax.ShapeDtypeStruct((_N_ROWS, _D), jnp.float32),
                   jax.ShapeDtypeStruct((_NW * _B_PER_W, _KP), jnp.int32),
                   jax.ShapeDtypeStruct((_NW * _B_PER_W, _D), jnp.float32)),
    )(step, wbuf, ibuf)


def _sc_gather_sum(table, ids, aux):
    mesh = plsc.VectorSubcoreMesh(core_axis_name="c", subcore_axis_name="s")

    @functools.partial(
        pl.kernel,
        mesh=mesh,
        out_type=jax.ShapeDtypeStruct((_NW * _B_PER_W, _D), jnp.float32),
        scratch_types=[
            pltpu.VMEM((_B_PER_W, _KP), jnp.int32),
            pltpu.VMEM((_B_PER_W, _K, _D), jnp.float32),
            pltpu.VMEM((_B_PER_W, _D), jnp.float32),
            pltpu.SemaphoreType.DMA,
            pltpu.SemaphoreType.DMA,
        ],
    )
    def k(table_hbm, ids_hbm, aux_hbm, out_hbm, idx_v, buf_v, acc_v, sem,
          sem2):
        wid = lax.axis_index("s") * 2 + lax.axis_index("c")
        row0 = wid * _B_PER_W
        pltpu.sync_copy(ids_hbm.at[pl.ds(row0, _B_PER_W)], idx_v)
        aux_cp = pltpu.async_copy(aux_hbm.at[pl.ds(row0, _B_PER_W)], acc_v,
                                  sem2)
        copies = [
            pltpu.async_copy(table_hbm.at[idx_v.at[o, pl.ds(0, _K)]],
                             buf_v.at[o], sem)
            for o in range(_B_PER_W)
        ]
        aux_cp.wait()
        for o in range(_B_PER_W):
            copies[o].wait()
            for v in range(_D // 16):
                sl = pl.ds(v * 16, 16)
                acc = acc_v[o, sl]
                for j in range(_K):
                    acc = acc + buf_v[o, j, sl]
                acc_v[o, sl] = acc
        pltpu.sync_copy(acc_v, out_hbm.at[pl.ds(row0, _B_PER_W)])

    return k(table, ids, aux)


def kernel(step, note_table, note_gate, transpose_table, transpose_gate,
           cmd_table, val_table, dummy_vec, fx_proj, table_proj,
           groove_tick_table, groove_proj, softsynths, waveframes, soft_proj,
           wave_proj, instr_proj, channel_projections, grooves, traces,
           tables, instruments):
    tab, ids, aux = _precompute_table(
        step.astype(jnp.int32).reshape(_NW * _B_PER_W, 21), note_table,
        note_gate, transpose_table, transpose_gate, cmd_table, val_table,
        dummy_vec, fx_proj, table_proj, groove_tick_table, groove_proj,
        softsynths, waveframes, soft_proj, wave_proj, instr_proj,
        channel_projections, grooves, traces, tables, instruments)
    out = _sc_gather_sum(tab, ids, aux)
    return out.reshape(256, 4, _D)
